# baseline (speedup 1.0000x reference)
import functools

import jax
import jax.numpy as jnp
from jax import lax
from jax.experimental import pallas as pl
from jax.experimental.pallas import tpu as pltpu
from jax.experimental.pallas import tpu_sc as plsc

B = 4096
L = 200
D = 300
DP = 304
OUTW = 912
H1 = 256
NUM_CLASSES = 1000

NC = 2
NS = 16
NW = NC * NS
BPW = B // NW
G = 8
C1, C2 = 104, 96

OFFS = tuple(range(0, DP, 16))
NACC = len(OFFS)


@functools.cache
def _get_pool():
    mesh = plsc.VectorSubcoreMesh(core_axis_name="c", subcore_axis_name="s")

    @functools.partial(
        pl.kernel,
        out_type=jax.ShapeDtypeStruct((B, OUTW), jnp.float32),
        mesh=mesh,
        scratch_types=[
            pltpu.VMEM((G * L,), jnp.int32),
            pltpu.VMEM((G * L,), jnp.int32),
            pltpu.VMEM((G * L,), jnp.int32),
            pltpu.VMEM((C1, DP), jnp.float32),
            pltpu.VMEM((C1, DP), jnp.float32),
            pltpu.VMEM((G, OUTW), jnp.float32),
            pltpu.SemaphoreType.DMA,
            pltpu.SemaphoreType.DMA,
        ],
        compiler_params=pltpu.CompilerParams(use_tc_tiling_on_sc=False),
    )
    def _pool(words_hbm, bigrams_hbm, trigrams_hbm, e1, e2, e3, out_hbm,
              widx, bidx, tidx, rows0, rows1, out_v, sem0, sem1):
        _pool_body(words_hbm, bigrams_hbm, trigrams_hbm, e1, e2, e3, out_hbm,
                   widx, bidx, tidx, rows0, rows1, out_v, sem0, sem1)

    return _pool


def _pool_body(words_hbm, bigrams_hbm, trigrams_hbm, e1, e2, e3, out_hbm,
               widx, bidx, tidx, rows0, rows1, out_v, sem0, sem1):
    wid = lax.axis_index("s") * NC + lax.axis_index("c")
    rows = (rows0, rows1)
    sems = (sem0, sem1)
    UNROLL = 2

    units = []
    for s in range(G):
        for t in range(3):
            units.append((s, t, s * L, C1))
            units.append((s, t, s * L + C1, C2))

    def fire(u, idx_refs, tabs):
        s, t, off, n = units[u]
        p = u % 2
        return pltpu.async_copy(
            tabs[t].at[idx_refs[t].at[pl.ds(off, n)]],
            rows[p].at[pl.ds(0, n)], sems[p])

    def accum(rows_ref, n, accs):
        def body(r, a):
            new = list(a)
            for rr in range(UNROLL):
                new = [new[i] + rows_ref[r * UNROLL + rr, pl.ds(OFFS[i], 16)]
                       for i in range(NACC)]
            return tuple(new)
        return lax.fori_loop(0, n // UNROLL, body, accs)

    def group_body(grp, carry):
        base = wid * BPW + grp * G
        base_el = pl.multiple_of(base * L, 8)
        pltpu.sync_copy(words_hbm.at[pl.ds(base_el, G * L)], widx)
        pltpu.sync_copy(bigrams_hbm.at[pl.ds(base_el, G * L)], bidx)
        pltpu.sync_copy(trigrams_hbm.at[pl.ds(base_el, G * L)], tidx)
        idx_refs = (widx, bidx, tidx)
        tabs = (e1, e2, e3)

        cps = [None] * len(units)
        cps[0] = fire(0, idx_refs, tabs)
        accs = tuple(jnp.zeros((16,), jnp.float32) for _ in range(NACC))
        for u, (s, t, off, n) in enumerate(units):
            if u + 1 < len(units):
                cps[u + 1] = fire(u + 1, idx_refs, tabs)
            cps[u].wait()
            accs = accum(rows[u % 2], n, accs)
            if n == C2:
                for i in range(NACC):
                    out_v[s, pl.ds(t * D + OFFS[i], 16)] = accs[i]
                accs = tuple(jnp.zeros((16,), jnp.float32)
                             for _ in range(NACC))
        pltpu.sync_copy(out_v, out_hbm.at[pl.ds(pl.multiple_of(base, 8), G)])
        return carry

    lax.fori_loop(0, BPW // G, group_body, 0)


V1 = 100000
V2 = 200000
BKP = 200


@functools.cache
def _get_padder():
    mesh = plsc.VectorSubcoreMesh(core_axis_name="c", subcore_axis_name="s")

    @functools.partial(
        pl.kernel,
        out_type=(jax.ShapeDtypeStruct((V1, DP), jnp.float32),
                  jax.ShapeDtypeStruct((V2, DP), jnp.float32),
                  jax.ShapeDtypeStruct((V2, DP), jnp.float32)),
        mesh=mesh,
        scratch_types=[
            pltpu.VMEM((BKP, DP), jnp.float32),
            pltpu.VMEM((BKP, DP), jnp.float32),
            pltpu.SemaphoreType.DMA,
            pltpu.SemaphoreType.DMA,
            pltpu.SemaphoreType.DMA,
            pltpu.SemaphoreType.DMA,
        ],
        compiler_params=pltpu.CompilerParams(use_tc_tiling_on_sc=False),
    )
    def _padder(e1, e2, e3, p1, p2, p3, buf0, buf1,
                si0, si1, so0, so1):
        wid = lax.axis_index("s") * NC + lax.axis_index("c")
        for r in range(BKP):
            buf0[r, pl.ds(DP - 16, 16)] = jnp.zeros((16,), jnp.float32)
            buf1[r, pl.ds(DP - 16, 16)] = jnp.zeros((16,), jnp.float32)

        def pad_one(src, dst):
            nblk = src.shape[0] // BKP
            nmine = nblk // NW + jnp.where(wid < nblk % NW, 1, 0)

            def body(i, carry):
                b0 = wid + NW * 2 * i
                b1 = b0 + NW
                r0 = pl.multiple_of(b0 * BKP, 8)
                r1 = pl.multiple_of(b1 * BKP, 8)
                have1 = 2 * i + 1 < nmine
                cin0 = pltpu.async_copy(
                    src.at[pl.ds(r0, BKP)], buf0.at[:, pl.ds(0, D)], si0)

                @pl.when(have1)
                def _():
                    pltpu.async_copy(
                        src.at[pl.ds(r1, BKP)], buf1.at[:, pl.ds(0, D)],
                        si1).wait()
                cin0.wait()
                cout0 = pltpu.async_copy(buf0, dst.at[pl.ds(r0, BKP)], so0)

                @pl.when(have1)
                def _():
                    pltpu.async_copy(buf1, dst.at[pl.ds(r1, BKP)],
                                     so1).wait()
                cout0.wait()
                return carry

            lax.fori_loop(0, (nmine + 1) // 2, body, 0)

        pad_one(e1, p1)
        pad_one(e2, p2)
        pad_one(e3, p3)

    return _padder


def _flat_idx_body(x_ref, o_ref):
    for k in range(x_ref.shape[0]):
        o_ref[pl.ds(k * L, L)] = x_ref[k, :]


def _flatten_idx(x):
    bm = 128
    return pl.pallas_call(
        _flat_idx_body,
        grid=(B // bm,),
        in_specs=[pl.BlockSpec((bm, L), lambda i: (i, 0))],
        out_specs=pl.BlockSpec((bm * L,), lambda i: (i,)),
        out_shape=jax.ShapeDtypeStruct((B * L,), jnp.int32),
    )(x)


def _mlp_body(x_ref, w1_ref, b1_ref, w2_ref, b2_ref, o_ref):
    x = x_ref[...] * (1.0 / L)
    h = jnp.dot(x, w1_ref[...], preferred_element_type=jnp.float32)
    h = jnp.maximum(h + b1_ref[...], 0.0)
    o = jnp.dot(h, w2_ref[...], preferred_element_type=jnp.float32)
    o_ref[...] = o + b2_ref[...]


def _mlp(pooled, fc1_w, fc1_b, fc2_w, fc2_b):
    bm = 512
    return pl.pallas_call(
        _mlp_body,
        grid=(B // bm,),
        in_specs=[
            pl.BlockSpec((bm, 3 * D), lambda i: (i, 0)),
            pl.BlockSpec((3 * D, H1), lambda i: (0, 0)),
            pl.BlockSpec((1, H1), lambda i: (0, 0)),
            pl.BlockSpec((H1, NUM_CLASSES), lambda i: (0, 0)),
            pl.BlockSpec((1, NUM_CLASSES), lambda i: (0, 0)),
        ],
        out_specs=pl.BlockSpec((bm, NUM_CLASSES), lambda i: (i, 0)),
        out_shape=jax.ShapeDtypeStruct((B, NUM_CLASSES), jnp.float32),
    )(pooled, fc1_w, fc1_b.reshape(1, H1), fc2_w, fc2_b.reshape(1, NUM_CLASSES))


def kernel(words, bigrams, trigrams, emb1, emb2, emb3,
           fc1_w, fc1_b, fc2_w, fc2_b):
    def padlin(e):
        v = e.shape[0]
        z = jnp.zeros((v, DP - D), jnp.float32)
        return jnp.concatenate([e, z], axis=1).reshape(v * DP).reshape(v, DP)

    pooled = _get_pool()(_flatten_idx(words.astype(jnp.int32)),
                         _flatten_idx(bigrams.astype(jnp.int32)),
                         _flatten_idx(trigrams.astype(jnp.int32)),
                         padlin(emb1), padlin(emb2), padlin(emb3))
    return _mlp(pooled[:, :3 * D], fc1_w, fc1_b, fc2_w, fc2_b)


# speedup vs baseline: 1.2096x; 1.2096x over previous
import functools

import jax
import jax.numpy as jnp
from jax import lax
from jax.experimental import pallas as pl
from jax.experimental.pallas import tpu as pltpu
from jax.experimental.pallas import tpu_sc as plsc

B = 4096
L = 200
D = 300
DP = 304
OUTW = 912
H1 = 256
NUM_CLASSES = 1000

NC = 2
NS = 16
NW = NC * NS
BPW = B // NW
G = 8
C1, C2 = 104, 96

OFFS = tuple(range(0, DP, 16))
NACC = len(OFFS)


@functools.cache
def _get_pool():
    mesh = plsc.VectorSubcoreMesh(core_axis_name="c", subcore_axis_name="s")

    @functools.partial(
        pl.kernel,
        out_type=jax.ShapeDtypeStruct((B, OUTW), jnp.float32),
        mesh=mesh,
        scratch_types=[
            pltpu.VMEM((G * L,), jnp.int32),
            pltpu.VMEM((G * L,), jnp.int32),
            pltpu.VMEM((G * L,), jnp.int32),
            pltpu.VMEM((C1, DP), jnp.float32),
            pltpu.VMEM((C1, DP), jnp.float32),
            pltpu.VMEM((G, OUTW), jnp.float32),
            pltpu.SemaphoreType.DMA,
            pltpu.SemaphoreType.DMA,
        ],
        compiler_params=pltpu.CompilerParams(use_tc_tiling_on_sc=False),
    )
    def _pool(words_hbm, bigrams_hbm, trigrams_hbm, e1, e2, e3, out_hbm,
              widx, bidx, tidx, rows0, rows1, out_v, sem0, sem1):
        _pool_body(words_hbm, bigrams_hbm, trigrams_hbm, e1, e2, e3, out_hbm,
                   widx, bidx, tidx, rows0, rows1, out_v, sem0, sem1)

    return _pool


def _pool_body(words_hbm, bigrams_hbm, trigrams_hbm, e1, e2, e3, out_hbm,
               widx, bidx, tidx, rows0, rows1, out_v, sem0, sem1):
    wid = lax.axis_index("s") * NC + lax.axis_index("c")
    rows = (rows0, rows1)
    sems = (sem0, sem1)
    UNROLL = 2

    units = []
    for s in range(G):
        for t in range(3):
            units.append((s, t, s * L, C1))
            units.append((s, t, s * L + C1, C2))

    def fire(u, idx_refs, tabs):
        s, t, off, n = units[u]
        p = u % 2
        return pltpu.async_copy(
            tabs[t].at[idx_refs[t].at[pl.ds(off, n)]],
            rows[p].at[pl.ds(0, n)], sems[p])

    def accum(rows_ref, n, accs):
        def body(r, a):
            new = list(a)
            for rr in range(UNROLL):
                new = [new[i] + rows_ref[r * UNROLL + rr, pl.ds(OFFS[i], 16)]
                       for i in range(NACC)]
            return tuple(new)
        return lax.fori_loop(0, n // UNROLL, body, accs)

    def group_body(grp, carry):
        base = wid * BPW + grp * G
        base_el = pl.multiple_of(base * L, 8)
        pltpu.sync_copy(words_hbm.at[pl.ds(base_el, G * L)], widx)
        pltpu.sync_copy(bigrams_hbm.at[pl.ds(base_el, G * L)], bidx)
        pltpu.sync_copy(trigrams_hbm.at[pl.ds(base_el, G * L)], tidx)
        idx_refs = (widx, bidx, tidx)
        tabs = (e1, e2, e3)

        cps = [None] * len(units)
        cps[0] = fire(0, idx_refs, tabs)
        accs = tuple(jnp.zeros((16,), jnp.float32) for _ in range(NACC))
        for u, (s, t, off, n) in enumerate(units):
            if u + 1 < len(units):
                cps[u + 1] = fire(u + 1, idx_refs, tabs)
            cps[u].wait()
            accs = accum(rows[u % 2], n, accs)
            if n == C2:
                for i in range(NACC):
                    out_v[s, pl.ds(t * D + OFFS[i], 16)] = accs[i]
                accs = tuple(jnp.zeros((16,), jnp.float32)
                             for _ in range(NACC))
        pltpu.sync_copy(out_v, out_hbm.at[pl.ds(pl.multiple_of(base, 8), G)])
        return carry

    lax.fori_loop(0, BPW // G, group_body, 0)


def _pad_body(x_ref, o_ref):
    o_ref[...] = jnp.concatenate(
        [x_ref[...], jnp.zeros((x_ref.shape[0], DP - D), jnp.float32)],
        axis=1)


def _pad_table(e):
    v = e.shape[0]
    bm = 800
    return pl.pallas_call(
        _pad_body,
        grid=(v // bm,),
        in_specs=[pl.BlockSpec((bm, D), lambda i: (i, 0))],
        out_specs=pl.BlockSpec((bm, DP), lambda i: (i, 0)),
        out_shape=jax.ShapeDtypeStruct((v, DP), jnp.float32),
    )(e)


V1 = 100000
V2 = 200000
BKP = 200


@functools.cache
def _get_padder():
    mesh = plsc.VectorSubcoreMesh(core_axis_name="c", subcore_axis_name="s")

    @functools.partial(
        pl.kernel,
        out_type=(jax.ShapeDtypeStruct((V1, DP), jnp.float32),
                  jax.ShapeDtypeStruct((V2, DP), jnp.float32),
                  jax.ShapeDtypeStruct((V2, DP), jnp.float32)),
        mesh=mesh,
        scratch_types=[
            pltpu.VMEM((BKP, DP), jnp.float32),
            pltpu.VMEM((BKP, DP), jnp.float32),
            pltpu.SemaphoreType.DMA,
            pltpu.SemaphoreType.DMA,
            pltpu.SemaphoreType.DMA,
            pltpu.SemaphoreType.DMA,
        ],
        compiler_params=pltpu.CompilerParams(use_tc_tiling_on_sc=False),
    )
    def _padder(e1, e2, e3, p1, p2, p3, buf0, buf1,
                si0, si1, so0, so1):
        wid = lax.axis_index("s") * NC + lax.axis_index("c")
        for r in range(BKP):
            buf0[r, pl.ds(DP - 16, 16)] = jnp.zeros((16,), jnp.float32)
            buf1[r, pl.ds(DP - 16, 16)] = jnp.zeros((16,), jnp.float32)

        def pad_one(src, dst):
            nblk = src.shape[0] // BKP
            nmine = nblk // NW + jnp.where(wid < nblk % NW, 1, 0)

            def body(i, carry):
                b0 = wid + NW * 2 * i
                b1 = b0 + NW
                r0 = pl.multiple_of(b0 * BKP, 8)
                r1 = pl.multiple_of(b1 * BKP, 8)
                have1 = 2 * i + 1 < nmine
                cin0 = pltpu.async_copy(
                    src.at[pl.ds(r0, BKP)], buf0.at[:, pl.ds(0, D)], si0)

                @pl.when(have1)
                def _():
                    pltpu.async_copy(
                        src.at[pl.ds(r1, BKP)], buf1.at[:, pl.ds(0, D)],
                        si1).wait()
                cin0.wait()
                cout0 = pltpu.async_copy(buf0, dst.at[pl.ds(r0, BKP)], so0)

                @pl.when(have1)
                def _():
                    pltpu.async_copy(buf1, dst.at[pl.ds(r1, BKP)],
                                     so1).wait()
                cout0.wait()
                return carry

            lax.fori_loop(0, (nmine + 1) // 2, body, 0)

        pad_one(e1, p1)
        pad_one(e2, p2)
        pad_one(e3, p3)

    return _padder


def _flat_idx_body(x_ref, o_ref):
    for k in range(x_ref.shape[0]):
        o_ref[pl.ds(k * L, L)] = x_ref[k, :]


def _flatten_idx(x):
    bm = 128
    return pl.pallas_call(
        _flat_idx_body,
        grid=(B // bm,),
        in_specs=[pl.BlockSpec((bm, L), lambda i: (i, 0))],
        out_specs=pl.BlockSpec((bm * L,), lambda i: (i,)),
        out_shape=jax.ShapeDtypeStruct((B * L,), jnp.int32),
    )(x)


def _mlp_body(x_ref, w1_ref, b1_ref, w2_ref, b2_ref, o_ref):
    x = x_ref[...] * (1.0 / L)
    h = jnp.dot(x, w1_ref[...], preferred_element_type=jnp.float32)
    h = jnp.maximum(h + b1_ref[...], 0.0)
    o = jnp.dot(h, w2_ref[...], preferred_element_type=jnp.float32)
    o_ref[...] = o + b2_ref[...]


def _mlp(pooled, fc1_w, fc1_b, fc2_w, fc2_b):
    bm = 512
    return pl.pallas_call(
        _mlp_body,
        grid=(B // bm,),
        in_specs=[
            pl.BlockSpec((bm, 3 * D), lambda i: (i, 0)),
            pl.BlockSpec((3 * D, H1), lambda i: (0, 0)),
            pl.BlockSpec((1, H1), lambda i: (0, 0)),
            pl.BlockSpec((H1, NUM_CLASSES), lambda i: (0, 0)),
            pl.BlockSpec((1, NUM_CLASSES), lambda i: (0, 0)),
        ],
        out_specs=pl.BlockSpec((bm, NUM_CLASSES), lambda i: (i, 0)),
        out_shape=jax.ShapeDtypeStruct((B, NUM_CLASSES), jnp.float32),
    )(pooled, fc1_w, fc1_b.reshape(1, H1), fc2_w, fc2_b.reshape(1, NUM_CLASSES))


def kernel(words, bigrams, trigrams, emb1, emb2, emb3,
           fc1_w, fc1_b, fc2_w, fc2_b):
    pooled = _get_pool()(_flatten_idx(words.astype(jnp.int32)),
                         _flatten_idx(bigrams.astype(jnp.int32)),
                         _flatten_idx(trigrams.astype(jnp.int32)),
                         _pad_table(emb1), _pad_table(emb2),
                         _pad_table(emb3))
    return _mlp(pooled[:, :3 * D], fc1_w, fc1_b, fc2_w, fc2_b)


# speedup vs baseline: 1.2229x; 1.0109x over previous
import functools

import jax
import jax.numpy as jnp
from jax import lax
from jax.experimental import pallas as pl
from jax.experimental.pallas import tpu as pltpu
from jax.experimental.pallas import tpu_sc as plsc

B = 4096
L = 200
D = 300
DP = 304
OUTW = 912
H1 = 256
NUM_CLASSES = 1000

NC = 2
NS = 16
NW = NC * NS
BPW = B // NW
G = 8
C1, C2 = 104, 96

OFFS = tuple(range(0, DP, 16))
NACC = len(OFFS)


@functools.cache
def _get_pool():
    mesh = plsc.VectorSubcoreMesh(core_axis_name="c", subcore_axis_name="s")

    @functools.partial(
        pl.kernel,
        out_type=jax.ShapeDtypeStruct((B, OUTW), jnp.float32),
        mesh=mesh,
        scratch_types=[
            pltpu.VMEM((G * L,), jnp.int32),
            pltpu.VMEM((G * L,), jnp.int32),
            pltpu.VMEM((G * L,), jnp.int32),
            pltpu.VMEM((C1, DP), jnp.float32),
            pltpu.VMEM((C1, DP), jnp.float32),
            pltpu.VMEM((G, OUTW), jnp.float32),
            pltpu.SemaphoreType.DMA,
            pltpu.SemaphoreType.DMA,
        ],
        compiler_params=pltpu.CompilerParams(use_tc_tiling_on_sc=False),
    )
    def _pool(words_hbm, bigrams_hbm, trigrams_hbm, e1, e2, e3, out_hbm,
              widx, bidx, tidx, rows0, rows1, out_v, sem0, sem1):
        _pool_body(words_hbm, bigrams_hbm, trigrams_hbm, e1, e2, e3, out_hbm,
                   widx, bidx, tidx, rows0, rows1, out_v, sem0, sem1)

    return _pool


def _pool_body(words_hbm, bigrams_hbm, trigrams_hbm, e1, e2, e3, out_hbm,
               widx, bidx, tidx, rows0, rows1, out_v, sem0, sem1):
    wid = lax.axis_index("s") * NC + lax.axis_index("c")
    rows = (rows0, rows1)
    sems = (sem0, sem1)
    UNROLL = 2

    units = []
    for s in range(G):
        for t in range(3):
            units.append((s, t, s * L, C1))
            units.append((s, t, s * L + C1, C2))

    def fire(u, idx_refs, tabs):
        s, t, off, n = units[u]
        p = u % 2
        return pltpu.async_copy(
            tabs[t].at[idx_refs[t].at[pl.ds(off, n)]],
            rows[p].at[pl.ds(0, n)], sems[p])

    def accum(rows_ref, n, accs):
        def body(r, a):
            new = list(a)
            for rr in range(UNROLL):
                new = [new[i] + rows_ref[r * UNROLL + rr, pl.ds(OFFS[i], 16)]
                       for i in range(NACC)]
            return tuple(new)
        return lax.fori_loop(0, n // UNROLL, body, accs)

    def group_body(grp, carry):
        base = wid * BPW + grp * G
        base_el = pl.multiple_of(base * L, 8)
        pltpu.sync_copy(words_hbm.at[pl.ds(base_el, G * L)], widx)
        pltpu.sync_copy(bigrams_hbm.at[pl.ds(base_el, G * L)], bidx)
        pltpu.sync_copy(trigrams_hbm.at[pl.ds(base_el, G * L)], tidx)
        idx_refs = (widx, bidx, tidx)
        tabs = (e1, e2, e3)

        cps = [None] * len(units)
        cps[0] = fire(0, idx_refs, tabs)
        accs = tuple(jnp.zeros((16,), jnp.float32) for _ in range(NACC))
        for u, (s, t, off, n) in enumerate(units):
            if u + 1 < len(units):
                cps[u + 1] = fire(u + 1, idx_refs, tabs)
            cps[u].wait()
            accs = accum(rows[u % 2], n, accs)
            if n == C2:
                for i in range(NACC):
                    out_v[s, pl.ds(t * D + OFFS[i], 16)] = accs[i]
                accs = tuple(jnp.zeros((16,), jnp.float32)
                             for _ in range(NACC))
        pltpu.sync_copy(out_v, out_hbm.at[pl.ds(pl.multiple_of(base, 8), G)])
        return carry

    lax.fori_loop(0, BPW // G, group_body, 0)


def _pad_body(x_ref, o_ref):
    o_ref[...] = jnp.concatenate(
        [x_ref[...], jnp.zeros((x_ref.shape[0], DP - D), jnp.float32)],
        axis=1)


def _pad_table(e):
    v = e.shape[0]
    bm = 800
    return pl.pallas_call(
        _pad_body,
        grid=(v // bm,),
        in_specs=[pl.BlockSpec((bm, D), lambda i: (i, 0))],
        out_specs=pl.BlockSpec((bm, DP), lambda i: (i, 0)),
        out_shape=jax.ShapeDtypeStruct((v, DP), jnp.float32),
    )(e)


def _mlp_body(x_ref, w1_ref, b1_ref, w2_ref, b2_ref, o_ref):
    x = x_ref[...] * (1.0 / L)
    h = jnp.dot(x, w1_ref[...], preferred_element_type=jnp.float32)
    h = jnp.maximum(h + b1_ref[...], 0.0)
    o = jnp.dot(h, w2_ref[...], preferred_element_type=jnp.float32)
    o_ref[...] = o + b2_ref[...]


def _mlp(pooled, fc1_w, fc1_b, fc2_w, fc2_b):
    bm = 512
    return pl.pallas_call(
        _mlp_body,
        grid=(B // bm,),
        in_specs=[
            pl.BlockSpec((bm, 3 * D), lambda i: (i, 0)),
            pl.BlockSpec((3 * D, H1), lambda i: (0, 0)),
            pl.BlockSpec((1, H1), lambda i: (0, 0)),
            pl.BlockSpec((H1, NUM_CLASSES), lambda i: (0, 0)),
            pl.BlockSpec((1, NUM_CLASSES), lambda i: (0, 0)),
        ],
        out_specs=pl.BlockSpec((bm, NUM_CLASSES), lambda i: (i, 0)),
        out_shape=jax.ShapeDtypeStruct((B, NUM_CLASSES), jnp.float32),
    )(pooled, fc1_w, fc1_b.reshape(1, H1), fc2_w, fc2_b.reshape(1, NUM_CLASSES))


def kernel(words, bigrams, trigrams, emb1, emb2, emb3,
           fc1_w, fc1_b, fc2_w, fc2_b):
    pooled = _get_pool()(words.astype(jnp.int32).reshape(B * L),
                         bigrams.astype(jnp.int32).reshape(B * L),
                         trigrams.astype(jnp.int32).reshape(B * L),
                         _pad_table(emb1), _pad_table(emb2),
                         _pad_table(emb3))
    return _mlp(pooled[:, :3 * D], fc1_w, fc1_b, fc2_w, fc2_b)
